# Initial kernel scaffold; baseline (speedup 1.0000x reference)
import jax
import jax.numpy as jnp
from jax.experimental import pallas as pl


def kernel(center_ids, context_ids, in_embed, out_embed):
    raise NotImplementedError("write your pallas kernel here")


# speedup vs baseline: 8.3904x; 8.3904x over previous
import functools

import jax
import jax.numpy as jnp
from jax import lax
from jax.experimental import pallas as pl
from jax.experimental.pallas import tpu as pltpu
from jax.experimental.pallas import tpu_sc as plsc

VOCAB = 100000
D = 64
B = 16384
CTX = 20

NC = 2
NS = 16
NW = NC * NS
BW = B // NW
G = 32
ROWS = G * CTX
NCH = ROWS // 128
NG = BW // G
PW = BW * CTX


def _sc_scores_body(cen_idx_hbm, ctx_idx_hbm, in_hbm, out_hbm, scores_hbm,
                    cen_idx_v, ctx_idx_v, cen_rows_v, ctx_rows_v, scores_v,
                    sem0, sem1):
    wid = lax.axis_index("s") * NC + lax.axis_index("c")

    pltpu.sync_copy(cen_idx_hbm.at[wid], cen_idx_v)
    pltpu.sync_copy(ctx_idx_hbm.at[wid], ctx_idx_v)

    sems = (sem0, sem1)

    def _descs(g, b):
        sem = sems[b]
        ds = []
        for k in range(NCH):
            ds.append(pltpu.make_async_copy(
                out_hbm.at[ctx_idx_v.at[g * NCH + k]],
                ctx_rows_v.at[b, pl.ds(k * 128, 128)],
                sem))
        ds.append(pltpu.make_async_copy(
            in_hbm.at[cen_idx_v.at[g]],
            cen_rows_v.at[b],
            sem))
        return ds

    def _issue(g, b):
        for d in _descs(g, b):
            d.start()

    def _wait(g, b):
        for d in _descs(g, b):
            d.wait()

    lane = lax.iota(jnp.int32, 16)
    mask15 = lane == 15

    def _compute(g, b):
        @pl.loop(0, G)
        def _item(i):
            cen = [cen_rows_v[b, i, pl.ds(16 * k, 16)] for k in range(4)]
            base = i * CTX
            for c in range(CTX):
                r = base + c
                p = ctx_rows_v[b, r, pl.ds(0, 16)] * cen[0]
                p += ctx_rows_v[b, r, pl.ds(16, 16)] * cen[1]
                p += ctx_rows_v[b, r, pl.ds(32, 16)] * cen[2]
                p += ctx_rows_v[b, r, pl.ds(48, 16)] * cen[3]
                ps = plsc.cumsum(p)
                idxv = jnp.full((16,), g * ROWS + r, jnp.int32)
                plsc.store_scatter(scores_v, [idxv], ps, mask=mask15)

    _issue(0, 0)
    _issue(1, 1)

    @pl.loop(0, NG, step=2)
    def _group(g):
        for b in range(2):
            gg = g + b
            _wait(gg, b)

            @pl.when(gg + 2 < NG)
            def _():
                _issue(gg + 2, b)

            _compute(gg, b)

    pltpu.sync_copy(scores_v, scores_hbm.at[wid])


def _tc_loss_body(s_ref, o_ref):
    x = s_ref[...]
    sp = jnp.log1p(jnp.exp(-x))
    o_ref[...] = (jnp.sum(sp) / jnp.float32(B * CTX)).reshape(1, 1)


@jax.jit
def kernel(center_ids, context_ids, in_embed, out_embed):
    cen_idx = center_ids.astype(jnp.int32).reshape(NW, NG, G)
    ctx_idx = context_ids.astype(jnp.int32).reshape(NW, NG * NCH, 128)

    mesh = plsc.VectorSubcoreMesh(core_axis_name="c", subcore_axis_name="s")
    scores = pl.kernel(
        _sc_scores_body,
        out_type=jax.ShapeDtypeStruct((NW, PW), jnp.float32),
        mesh=mesh,
        compiler_params=pltpu.CompilerParams(
            needs_layout_passes=False, use_tc_tiling_on_sc=False),
        scratch_types=[
            pltpu.VMEM((NG, G), jnp.int32),
            pltpu.VMEM((NG * NCH, 128), jnp.int32),
            pltpu.VMEM((2, G, D), jnp.float32),
            pltpu.VMEM((2, ROWS, D), jnp.float32),
            pltpu.VMEM((PW,), jnp.float32),
            pltpu.SemaphoreType.DMA,
            pltpu.SemaphoreType.DMA,
        ],
    )(cen_idx, ctx_idx, in_embed, out_embed)

    loss = pl.pallas_call(
        _tc_loss_body,
        out_shape=jax.ShapeDtypeStruct((1, 1), jnp.float32),
    )(scores.reshape(B * CTX // 128, 128))
    return loss[0, 0]


# speedup vs baseline: 11.7070x; 1.3953x over previous
import functools

import jax
import jax.numpy as jnp
from jax import lax
from jax.experimental import pallas as pl
from jax.experimental.pallas import tpu as pltpu
from jax.experimental.pallas import tpu_sc as plsc

VOCAB = 100000
D = 64
B = 16384
CTX = 20

NC = 2
NS = 16
NW = NC * NS
BW = B // NW
G = 32
ROWS = G * CTX
NCH = ROWS // 128
NG = BW // G
PW = BW * CTX


def _sc_scores_body(cen_idx_hbm, ctx_idx_hbm, in_hbm, out_hbm, scores_hbm,
                    cen_idx_v, ctx_idx_v, cen_rows_v, ctx_rows_v, scores_v,
                    sem0, sem1):
    wid = lax.axis_index("s") * NC + lax.axis_index("c")

    pltpu.sync_copy(cen_idx_hbm.at[wid], cen_idx_v)
    pltpu.sync_copy(ctx_idx_hbm.at[wid], ctx_idx_v)

    sems = (sem0, sem1)

    def _descs(g, b):
        sem = sems[b]
        ds = []
        for k in range(NCH):
            ds.append(pltpu.make_async_copy(
                out_hbm.at[ctx_idx_v.at[g * NCH + k]],
                ctx_rows_v.at[b, pl.ds(k * 128, 128)],
                sem))
        ds.append(pltpu.make_async_copy(
            in_hbm.at[cen_idx_v.at[g]],
            cen_rows_v.at[b],
            sem))
        return ds

    def _issue(g, b):
        for d in _descs(g, b):
            d.start()

    def _wait(g, b):
        for d in _descs(g, b):
            d.wait()

    lane = lax.iota(jnp.int32, 16)
    mask15 = lane == 15
    lane_m15 = lane - 15

    def _compute(g, b):
        @plsc.parallel_loop(0, G, unroll=2)
        def _item(i):
            cen = [cen_rows_v[b, i, pl.ds(16 * k, 16)] for k in range(4)]
            base = i * CTX
            idx0 = lane_m15 + (g * ROWS + base)
            for c in range(CTX):
                r = base + c
                p = ctx_rows_v[b, r, pl.ds(0, 16)] * cen[0]
                p += ctx_rows_v[b, r, pl.ds(16, 16)] * cen[1]
                p += ctx_rows_v[b, r, pl.ds(32, 16)] * cen[2]
                p += ctx_rows_v[b, r, pl.ds(48, 16)] * cen[3]
                ps = plsc.cumsum(p)
                plsc.store_scatter(scores_v, [idx0 + c], ps, mask=mask15)

    _issue(0, 0)
    _issue(1, 1)

    @pl.loop(0, NG, step=2)
    def _group(g):
        for b in range(2):
            gg = g + b
            _wait(gg, b)

            @pl.when(gg + 2 < NG)
            def _():
                _issue(gg + 2, b)

            _compute(gg, b)

    pltpu.sync_copy(scores_v, scores_hbm.at[wid])


def _tc_loss_body(s_ref, o_ref):
    x = s_ref[...]
    sp = jnp.log1p(jnp.exp(-x))
    o_ref[...] = (jnp.sum(sp) / jnp.float32(B * CTX)).reshape(1, 1)


@jax.jit
def kernel(center_ids, context_ids, in_embed, out_embed):
    cen_idx = center_ids.astype(jnp.int32).reshape(NW, NG, G)
    ctx_idx = context_ids.astype(jnp.int32).reshape(NW, NG * NCH, 128)

    mesh = plsc.VectorSubcoreMesh(core_axis_name="c", subcore_axis_name="s")
    scores = pl.kernel(
        _sc_scores_body,
        out_type=jax.ShapeDtypeStruct((NW, PW), jnp.float32),
        mesh=mesh,
        compiler_params=pltpu.CompilerParams(
            needs_layout_passes=False, use_tc_tiling_on_sc=False),
        scratch_types=[
            pltpu.VMEM((NG, G), jnp.int32),
            pltpu.VMEM((NG * NCH, 128), jnp.int32),
            pltpu.VMEM((2, G, D), jnp.float32),
            pltpu.VMEM((2, ROWS, D), jnp.float32),
            pltpu.VMEM((PW,), jnp.float32),
            pltpu.SemaphoreType.DMA,
            pltpu.SemaphoreType.DMA,
        ],
    )(cen_idx, ctx_idx, in_embed, out_embed)

    loss = pl.pallas_call(
        _tc_loss_body,
        out_shape=jax.ShapeDtypeStruct((1, 1), jnp.float32),
    )(scores.reshape(B * CTX // 128, 128))
    return loss[0, 0]
